# Initial kernel scaffold; baseline (speedup 1.0000x reference)
import jax
import jax.numpy as jnp
from jax.experimental import pallas as pl


def kernel(inputs, pos_table):
    raise NotImplementedError("write your pallas kernel here")


# speedup vs baseline: 2.8654x; 2.8654x over previous
import jax
import jax.numpy as jnp
from jax.experimental import pallas as pl


_SEQ_BLK = 512


def _add_kernel(x_ref, p_ref, o_ref):
    o_ref[...] = x_ref[...] + p_ref[...]


def kernel(inputs, pos_table):
    batch, seq_len, d_model = inputs.shape
    n_seq = seq_len // _SEQ_BLK
    return pl.pallas_call(
        _add_kernel,
        grid=(n_seq, batch),
        in_specs=[
            pl.BlockSpec((1, _SEQ_BLK, d_model), lambda i, j: (j, i, 0)),
            pl.BlockSpec((_SEQ_BLK, d_model), lambda i, j: (i, 0)),
        ],
        out_specs=pl.BlockSpec((1, _SEQ_BLK, d_model), lambda i, j: (j, i, 0)),
        out_shape=jax.ShapeDtypeStruct(inputs.shape, inputs.dtype),
    )(inputs, pos_table)


# speedup vs baseline: 3.1582x; 1.1022x over previous
import jax
import jax.numpy as jnp
from jax.experimental import pallas as pl


_SEQ_BLK = 1024


def _add_kernel(x_ref, p_ref, o_ref):
    o_ref[...] = x_ref[...] + p_ref[...]


def kernel(inputs, pos_table):
    batch, seq_len, d_model = inputs.shape
    n_seq = seq_len // _SEQ_BLK
    return pl.pallas_call(
        _add_kernel,
        grid=(n_seq, batch),
        in_specs=[
            pl.BlockSpec((1, _SEQ_BLK, d_model), lambda i, j: (j, i, 0)),
            pl.BlockSpec((_SEQ_BLK, d_model), lambda i, j: (i, 0)),
        ],
        out_specs=pl.BlockSpec((1, _SEQ_BLK, d_model), lambda i, j: (j, i, 0)),
        out_shape=jax.ShapeDtypeStruct(inputs.shape, inputs.dtype),
    )(inputs, pos_table)


# speedup vs baseline: 3.4040x; 1.0778x over previous
import jax
import jax.numpy as jnp
from jax.experimental import pallas as pl


_SEQ_BLK = 2048


def _add_kernel(x_ref, p_ref, o_ref):
    o_ref[...] = x_ref[...] + p_ref[...]


def kernel(inputs, pos_table):
    batch, seq_len, d_model = inputs.shape
    n_seq = seq_len // _SEQ_BLK
    return pl.pallas_call(
        _add_kernel,
        grid=(n_seq, batch),
        in_specs=[
            pl.BlockSpec((1, _SEQ_BLK, d_model), lambda i, j: (j, i, 0)),
            pl.BlockSpec((_SEQ_BLK, d_model), lambda i, j: (i, 0)),
        ],
        out_specs=pl.BlockSpec((1, _SEQ_BLK, d_model), lambda i, j: (j, i, 0)),
        out_shape=jax.ShapeDtypeStruct(inputs.shape, inputs.dtype),
    )(inputs, pos_table)
